# baseline (device time: 283833 ns/iter reference)
import jax
import jax.numpy as jnp
from jax import lax
from jax.experimental import pallas as pl
from jax.experimental.pallas import tpu as pltpu

WORLD = 16
CAP_D = 208
BLK = 224
CAP_E = 512


def _dispatch_body(send_ref, x_ref, sw_ref,
                   shared_out_ref, recv_ref, dsend, drecv):
    me = lax.axis_index("i")
    for k in range(1, WORLD):
        t = lax.rem(me + k, WORLD)
        pltpu.make_async_remote_copy(
            src_ref=send_ref.at[pl.ds(t * BLK, BLK)],
            dst_ref=recv_ref.at[pl.ds(me * BLK, BLK)],
            send_sem=dsend.at[t],
            recv_sem=drecv.at[me],
            device_id=(t,),
            device_id_type=pltpu.DeviceIdType.MESH,
        ).start()
    recv_ref[pl.ds(me * BLK, BLK)] = send_ref[pl.ds(me * BLK, BLK)]

    shared_out_ref[...] = jnp.dot(
        x_ref[...], sw_ref[...],
        preferred_element_type=jnp.float32).astype(shared_out_ref.dtype)

    for k in range(1, WORLD):
        s = lax.rem(me + k, WORLD)
        pltpu.make_async_remote_copy(
            src_ref=send_ref.at[pl.ds(s * BLK, BLK)],
            dst_ref=recv_ref.at[pl.ds(s * BLK, BLK)],
            send_sem=dsend.at[s], recv_sem=drecv.at[s],
            device_id=(s,), device_id_type=pltpu.DeviceIdType.MESH,
        ).wait_recv()
    for k in range(1, WORLD):
        t = lax.rem(me + k, WORLD)
        pltpu.make_async_remote_copy(
            src_ref=send_ref.at[pl.ds(t * BLK, BLK)],
            dst_ref=recv_ref.at[pl.ds(t * BLK, BLK)],
            send_sem=dsend.at[t], recv_sem=drecv.at[t],
            device_id=(t,), device_id_type=pltpu.DeviceIdType.MESH,
        ).wait_send()


def _compute_body(comp_ref, ew_ref, y_ref):
    n_le = ew_ref.shape[0]
    for le in range(n_le):
        y_ref[le * CAP_E:(le + 1) * CAP_E] = jnp.dot(
            comp_ref[le * CAP_E:(le + 1) * CAP_E, :], ew_ref[le],
            preferred_element_type=jnp.float32).astype(y_ref.dtype)


def _combine_body(cret_ref, retbuf_ref, csend, crecv):
    me = lax.axis_index("i")
    for k in range(1, WORLD):
        t = lax.rem(me + k, WORLD)
        pltpu.make_async_remote_copy(
            src_ref=cret_ref.at[pl.ds(t * CAP_D, CAP_D)],
            dst_ref=retbuf_ref.at[pl.ds(me * CAP_D, CAP_D)],
            send_sem=csend.at[t],
            recv_sem=crecv.at[me],
            device_id=(t,),
            device_id_type=pltpu.DeviceIdType.MESH,
        ).start()
    retbuf_ref[pl.ds(me * CAP_D, CAP_D)] = cret_ref[pl.ds(me * CAP_D, CAP_D)]
    for k in range(1, WORLD):
        s = lax.rem(me + k, WORLD)
        pltpu.make_async_remote_copy(
            src_ref=cret_ref.at[pl.ds(s * CAP_D, CAP_D)],
            dst_ref=retbuf_ref.at[pl.ds(s * CAP_D, CAP_D)],
            send_sem=csend.at[s], recv_sem=crecv.at[s],
            device_id=(s,), device_id_type=pltpu.DeviceIdType.MESH,
        ).wait_recv()
    for k in range(1, WORLD):
        t = lax.rem(me + k, WORLD)
        pltpu.make_async_remote_copy(
            src_ref=cret_ref.at[pl.ds(t * CAP_D, CAP_D)],
            dst_ref=retbuf_ref.at[pl.ds(me * CAP_D, CAP_D)],
            send_sem=csend.at[t], recv_sem=crecv.at[t],
            device_id=(t,), device_id_type=pltpu.DeviceIdType.MESH,
        ).wait_send()


def _rank_in_group(group, n_groups):
    oh = (group[:, None] == jnp.arange(n_groups, dtype=group.dtype)
          ).astype(jnp.int32)
    return jnp.sum(jnp.cumsum(oh, axis=0) * oh, axis=1) - 1


def kernel(x, router_W, route_idx, expert_W, shared_W):
    n_tok, d = x.shape
    n_exp = router_W.shape[1]
    n_le, _, h = expert_W.shape

    scores = x @ router_W
    m = jnp.max(scores, axis=-1, keepdims=True)
    p = 1.0 / jnp.sum(jnp.exp(scores - m), axis=-1, keepdims=True)
    e = route_idx[:, 0]
    xp = (x * p).astype(jnp.bfloat16)

    dst = e // n_le
    le = e % n_le
    pd = _rank_in_group(dst, WORLD)
    validt = pd < CAP_D
    pd_safe = jnp.where(validt, pd, BLK)

    send3d = jnp.zeros((WORLD, BLK, d), jnp.bfloat16)
    send3d = send3d.at[dst, pd_safe].set(xp, mode="drop")
    meta = jnp.full((WORLD, CAP_D), n_le, jnp.bfloat16)
    meta = meta.at[dst, pd_safe].set(le.astype(jnp.bfloat16), mode="drop")
    send3d = send3d.at[:, CAP_D, :CAP_D].set(meta)

    shared_out, recv = pl.pallas_call(
        _dispatch_body,
        out_shape=(
            jax.ShapeDtypeStruct((n_tok, h), jnp.bfloat16),
            jax.ShapeDtypeStruct((WORLD * BLK, d), jnp.bfloat16),
        ),
        in_specs=[pl.BlockSpec(memory_space=pltpu.VMEM)] * 3,
        out_specs=(pl.BlockSpec(memory_space=pltpu.VMEM),) * 2,
        scratch_shapes=[
            pltpu.SemaphoreType.DMA((WORLD,)),
            pltpu.SemaphoreType.DMA((WORLD,)),
        ],
        compiler_params=pltpu.CompilerParams(
            vmem_limit_bytes=63 * 1024 * 1024,
        ),
    )(send3d.reshape(WORLD * BLK, d), x.astype(jnp.bfloat16),
      shared_W.astype(jnp.bfloat16))

    recv3d = recv.reshape(WORLD, BLK, d)
    les = recv3d[:, CAP_D, :CAP_D].astype(jnp.int32).reshape(-1)
    rows = recv3d[:, :CAP_D, :].reshape(WORLD * CAP_D, d)
    rvalid = les < n_le
    pos = _rank_in_group(jnp.where(rvalid, les, n_le), n_le + 1)
    comp_idx = jnp.where(rvalid & (pos < CAP_E),
                         les * CAP_E + pos, n_le * CAP_E)
    comp = jnp.zeros((n_le * CAP_E, d), jnp.bfloat16).at[comp_idx].set(
        rows, mode="drop")

    y = pl.pallas_call(
        _compute_body,
        out_shape=jax.ShapeDtypeStruct((n_le * CAP_E, h), jnp.bfloat16),
        in_specs=[pl.BlockSpec(memory_space=pltpu.VMEM)] * 2,
        out_specs=pl.BlockSpec(memory_space=pltpu.VMEM),
        compiler_params=pltpu.CompilerParams(
            vmem_limit_bytes=63 * 1024 * 1024,
        ),
    )(comp, expert_W.astype(jnp.bfloat16))

    yret = jnp.take(y, jnp.where(rvalid, comp_idx, 0), axis=0)
    yret = yret * rvalid[:, None].astype(yret.dtype)

    retbuf = pl.pallas_call(
        _combine_body,
        out_shape=jax.ShapeDtypeStruct((WORLD * CAP_D, h), jnp.bfloat16),
        in_specs=[pl.BlockSpec(memory_space=pltpu.VMEM)],
        out_specs=pl.BlockSpec(memory_space=pltpu.VMEM),
        scratch_shapes=[
            pltpu.SemaphoreType.DMA((WORLD,)),
            pltpu.SemaphoreType.DMA((WORLD,)),
        ],
        compiler_params=pltpu.CompilerParams(
            vmem_limit_bytes=63 * 1024 * 1024,
        ),
    )(yret)

    taken = jnp.take(retbuf, jnp.where(validt, dst * CAP_D + pd, 0), axis=0)
    out = shared_out.astype(jnp.float32) + jnp.where(
        validt[:, None], taken.astype(jnp.float32), 0.0)
    return out


# device time: 272703 ns/iter; 1.0408x vs baseline; 1.0408x over previous
import jax
import jax.numpy as jnp
from jax import lax
from jax.experimental import pallas as pl
from jax.experimental.pallas import tpu as pltpu

WORLD = 16
CAP_D = 208
BLK = 224
CAP_E = 512


def _dispatch_body(send_ref, x_ref, sw_ref,
                   shared_out_ref, recv_ref, dsend, drecv):
    me = lax.axis_index("i")
    for k in range(1, WORLD):
        t = lax.rem(me + k, WORLD)
        pltpu.make_async_remote_copy(
            src_ref=send_ref.at[pl.ds(t * BLK, BLK)],
            dst_ref=recv_ref.at[pl.ds(me * BLK, BLK)],
            send_sem=dsend.at[t],
            recv_sem=drecv.at[me],
            device_id=(t,),
            device_id_type=pltpu.DeviceIdType.MESH,
        ).start()
    recv_ref[pl.ds(me * BLK, BLK)] = send_ref[pl.ds(me * BLK, BLK)]

    shared_out_ref[...] = jnp.dot(
        x_ref[...], sw_ref[...],
        preferred_element_type=jnp.float32).astype(shared_out_ref.dtype)

    for k in range(1, WORLD):
        s = lax.rem(me + k, WORLD)
        pltpu.make_async_remote_copy(
            src_ref=send_ref.at[pl.ds(s * BLK, BLK)],
            dst_ref=recv_ref.at[pl.ds(s * BLK, BLK)],
            send_sem=dsend.at[s], recv_sem=drecv.at[s],
            device_id=(s,), device_id_type=pltpu.DeviceIdType.MESH,
        ).wait_recv()
    for k in range(1, WORLD):
        t = lax.rem(me + k, WORLD)
        pltpu.make_async_remote_copy(
            src_ref=send_ref.at[pl.ds(t * BLK, BLK)],
            dst_ref=recv_ref.at[pl.ds(t * BLK, BLK)],
            send_sem=dsend.at[t], recv_sem=drecv.at[t],
            device_id=(t,), device_id_type=pltpu.DeviceIdType.MESH,
        ).wait_send()


def _compute_body(comp_ref, ew_ref, y_ref):
    n_le = ew_ref.shape[0]
    for le in range(n_le):
        y_ref[le * CAP_E:(le + 1) * CAP_E] = jnp.dot(
            comp_ref[le * CAP_E:(le + 1) * CAP_E, :],
            ew_ref[le].astype(jnp.bfloat16),
            preferred_element_type=jnp.float32).astype(y_ref.dtype)


def _combine_body(cret_ref, retbuf_ref, csend, crecv):
    me = lax.axis_index("i")
    for k in range(1, WORLD):
        t = lax.rem(me + k, WORLD)
        pltpu.make_async_remote_copy(
            src_ref=cret_ref.at[pl.ds(t * CAP_D, CAP_D)],
            dst_ref=retbuf_ref.at[pl.ds(me * CAP_D, CAP_D)],
            send_sem=csend.at[t],
            recv_sem=crecv.at[me],
            device_id=(t,),
            device_id_type=pltpu.DeviceIdType.MESH,
        ).start()
    retbuf_ref[pl.ds(me * CAP_D, CAP_D)] = cret_ref[pl.ds(me * CAP_D, CAP_D)]
    for k in range(1, WORLD):
        s = lax.rem(me + k, WORLD)
        pltpu.make_async_remote_copy(
            src_ref=cret_ref.at[pl.ds(s * CAP_D, CAP_D)],
            dst_ref=retbuf_ref.at[pl.ds(s * CAP_D, CAP_D)],
            send_sem=csend.at[s], recv_sem=crecv.at[s],
            device_id=(s,), device_id_type=pltpu.DeviceIdType.MESH,
        ).wait_recv()
    for k in range(1, WORLD):
        t = lax.rem(me + k, WORLD)
        pltpu.make_async_remote_copy(
            src_ref=cret_ref.at[pl.ds(t * CAP_D, CAP_D)],
            dst_ref=retbuf_ref.at[pl.ds(me * CAP_D, CAP_D)],
            send_sem=csend.at[t], recv_sem=crecv.at[t],
            device_id=(t,), device_id_type=pltpu.DeviceIdType.MESH,
        ).wait_send()


def _rank_in_group(group, n_groups):
    oh = (group[:, None] == jnp.arange(n_groups, dtype=group.dtype)
          ).astype(jnp.int32)
    return jnp.sum(jnp.cumsum(oh, axis=0) * oh, axis=1) - 1


def kernel(x, router_W, route_idx, expert_W, shared_W):
    n_tok, d = x.shape
    n_exp = router_W.shape[1]
    n_le, _, h = expert_W.shape

    scores = x @ router_W
    m = jnp.max(scores, axis=-1, keepdims=True)
    p = 1.0 / jnp.sum(jnp.exp(scores - m), axis=-1, keepdims=True)
    e = route_idx[:, 0]
    xp = (x * p).astype(jnp.bfloat16)

    dst = e // n_le
    le = e % n_le
    pd = _rank_in_group(dst, WORLD)
    validt = pd < CAP_D
    pd_safe = jnp.where(validt, pd, BLK)

    send3d = jnp.zeros((WORLD, BLK, d), jnp.bfloat16)
    send3d = send3d.at[dst, pd_safe].set(xp, mode="drop",
                                         unique_indices=True)
    meta = jnp.full((WORLD, CAP_D), n_le, jnp.bfloat16)
    meta = meta.at[dst, pd_safe].set(le.astype(jnp.bfloat16), mode="drop",
                                     unique_indices=True)
    send3d = send3d.at[:, CAP_D, :CAP_D].set(meta)

    shared_out, recv = pl.pallas_call(
        _dispatch_body,
        out_shape=(
            jax.ShapeDtypeStruct((n_tok, h), jnp.bfloat16),
            jax.ShapeDtypeStruct((WORLD * BLK, d), jnp.bfloat16),
        ),
        in_specs=[pl.BlockSpec(memory_space=pltpu.VMEM)] * 3,
        out_specs=(pl.BlockSpec(memory_space=pltpu.VMEM),) * 2,
        scratch_shapes=[
            pltpu.SemaphoreType.DMA((WORLD,)),
            pltpu.SemaphoreType.DMA((WORLD,)),
        ],
        compiler_params=pltpu.CompilerParams(
            vmem_limit_bytes=63 * 1024 * 1024,
        ),
    )(send3d.reshape(WORLD * BLK, d), x.astype(jnp.bfloat16),
      shared_W.astype(jnp.bfloat16))

    recv3d = recv.reshape(WORLD, BLK, d)
    les = recv3d[:, CAP_D, :CAP_D].astype(jnp.int32).reshape(-1)
    rows = recv3d[:, :CAP_D, :].reshape(WORLD * CAP_D, d)
    rvalid = les < n_le
    pos = _rank_in_group(jnp.where(rvalid, les, n_le), n_le + 1)
    comp_idx = jnp.where(rvalid & (pos < CAP_E),
                         les * CAP_E + pos, n_le * CAP_E)
    comp = jnp.zeros((n_le * CAP_E, d), jnp.bfloat16).at[comp_idx].set(
        rows, mode="drop", unique_indices=True)

    y = pl.pallas_call(
        _compute_body,
        out_shape=jax.ShapeDtypeStruct((n_le * CAP_E, h), jnp.bfloat16),
        in_specs=[pl.BlockSpec(memory_space=pltpu.VMEM)] * 2,
        out_specs=pl.BlockSpec(memory_space=pltpu.VMEM),
        compiler_params=pltpu.CompilerParams(
            vmem_limit_bytes=63 * 1024 * 1024,
        ),
    )(comp, expert_W)

    yret = jnp.take(y, comp_idx, axis=0, mode="clip")

    retbuf = pl.pallas_call(
        _combine_body,
        out_shape=jax.ShapeDtypeStruct((WORLD * CAP_D, h), jnp.bfloat16),
        in_specs=[pl.BlockSpec(memory_space=pltpu.VMEM)],
        out_specs=pl.BlockSpec(memory_space=pltpu.VMEM),
        scratch_shapes=[
            pltpu.SemaphoreType.DMA((WORLD,)),
            pltpu.SemaphoreType.DMA((WORLD,)),
        ],
        compiler_params=pltpu.CompilerParams(
            vmem_limit_bytes=63 * 1024 * 1024,
        ),
    )(yret)

    taken = jnp.take(retbuf, jnp.where(validt, dst * CAP_D + pd, 0), axis=0)
    out = shared_out.astype(jnp.float32) + jnp.where(
        validt[:, None], taken.astype(jnp.float32), 0.0)
    return out
